# baseline (speedup 1.0000x reference)
import functools

import jax
import jax.numpy as jnp
from jax import lax
from jax.experimental import pallas as pl
from jax.experimental.pallas import tpu as pltpu
from jax.experimental.pallas import tpu_sc as plsc

NB = 1024
S = 50
D = 1000
W = 512
G = 52
QB = 2 * G
SR = QB * (W // 128)
TAIL = D - 7 * 128
NC, NS_ = 2, 16
NW = NC * NS_
BPW = NB // NW

_mesh = plsc.VectorSubcoreMesh(core_axis_name="c", subcore_axis_name="s")


@functools.partial(
    pl.kernel,
    mesh=_mesh,
    out_type=jax.ShapeDtypeStruct((NB, S, D), jnp.float32),
    scratch_types=[
        pltpu.VMEM((BPW * QB,), jnp.int32),
        pltpu.VMEM((SR, 128), jnp.float32),
        pltpu.VMEM((SR, 128), jnp.float32),
        pltpu.VMEM((S, TAIL), jnp.float32),
        pltpu.VMEM((S, TAIL), jnp.float32),
        pltpu.SemaphoreType.DMA,
        pltpu.SemaphoreType.DMA,
    ],
)
def _gather_kernel(idx_hbm, w2_hbm, out_hbm, idx_v, stage_a, stage_b,
                   slab_a, slab_b, gsem, wsem):
    wid = lax.axis_index("s") * NC + lax.axis_index("c")
    b0 = wid * BPW
    pltpu.sync_copy(idx_hbm.at[pl.ds(b0 * QB, BPW * QB)], idx_v)

    def g_copies(k, stage):
        return [(w2_hbm.at[idx_v.at[pl.ds(k * QB, QB)]],
                 stage.reshape(QB, W))]

    def w_copies(k, stage, slab):
        b = b0 + k
        view = stage.reshape(QB, W)
        return [
            (view.at[pl.ds(0, S)], out_hbm.at[b, pl.ds(0, S), pl.ds(0, W)]),
            (view.at[pl.ds(G, S), pl.ds(0, 384)],
             out_hbm.at[b, pl.ds(0, S), pl.ds(W, 384)]),
            (slab, out_hbm.at[b, pl.ds(0, S), pl.ds(7 * 128, TAIL)]),
        ]

    def issue(cps, sem):
        for src, dst in cps:
            pltpu.async_copy(src, dst, sem)

    def drain(cps, sem):
        for src, dst in cps:
            pltpu.make_async_copy(src, dst, sem).wait()

    def repack(stage, slab):
        def row_body(s, c2):
            r = 4 * (G + s) + 3
            for c in range(6):
                slab[s, pl.ds(c * 16, 16)] = stage[r, pl.ds(c * 16, 16)]
            slab[s, pl.ds(TAIL - 16, 16)] = stage[r, pl.ds(TAIL - 16, 16)]
            return c2
        lax.fori_loop(0, S, row_body, 0)

    def step(k, cur, curslab, other, otherslab):
        drain(g_copies(k, cur), gsem)
        drain(w_copies(k - 1, other, otherslab), wsem)
        issue(g_copies(k + 1, other), gsem)
        repack(cur, curslab)
        issue(w_copies(k, cur, curslab), wsem)

    issue(g_copies(0, stage_a), gsem)
    drain(g_copies(0, stage_a), gsem)
    issue(g_copies(1, stage_b), gsem)
    repack(stage_a, slab_a)
    issue(w_copies(0, stage_a, slab_a), wsem)

    def mid(i, carry):
        k1 = 2 * i + 1
        step(k1, stage_b, slab_b, stage_a, slab_a)
        step(k1 + 1, stage_a, slab_a, stage_b, slab_b)
        return carry

    lax.fori_loop(0, (BPW - 2) // 2, mid, 0)

    k = BPW - 1
    drain(g_copies(k, stage_b), gsem)
    drain(w_copies(k - 1, stage_a, slab_a), wsem)
    repack(stage_b, slab_b)
    issue(w_copies(k, stage_b, slab_b), wsem)
    drain(w_copies(k, stage_b, slab_b), wsem)


def kernel(idx, w):
    w2 = jnp.pad(w, ((0, 0), (0, 24))).reshape(2 * 1000, W)
    base = jnp.pad(idx.astype(jnp.int32), ((0, 0), (0, G - S)))
    idx2 = (base[:, None, :] * 2
            + jnp.arange(2, dtype=jnp.int32)[None, :, None]).reshape(-1)
    return _gather_kernel(idx2, w2)


# speedup vs baseline: 1.1461x; 1.1461x over previous
import functools

import jax
import jax.numpy as jnp
from jax import lax
from jax.experimental import pallas as pl
from jax.experimental.pallas import tpu as pltpu
from jax.experimental.pallas import tpu_sc as plsc

NB = 1024
S = 50
D = 1000
DPW = 32
NC, NS_ = 2, 16
NW = NC * NS_

_mesh = plsc.VectorSubcoreMesh(core_axis_name="c", subcore_axis_name="s")


@functools.partial(
    pl.kernel,
    mesh=_mesh,
    out_type=jax.ShapeDtypeStruct((S, D, NB), jnp.float32),
    scratch_types=[
        pltpu.VMEM((DPW * 8, 128), jnp.float32),
        pltpu.VMEM((DPW * 8, 128), jnp.float32),
        pltpu.VMEM((DPW * 8, 128), jnp.float32),
        pltpu.VMEM((NB,), jnp.int32),
        pltpu.VMEM((NB,), jnp.int32),
        pltpu.SemaphoreType.DMA,
        pltpu.SemaphoreType.DMA,
        pltpu.SemaphoreType.DMA,
    ],
    compiler_params=pltpu.CompilerParams(needs_layout_passes=False),
)
def _lane_kernel(idxT_hbm, wT_hbm, r_hbm, wbuf, obuf_a, obuf_b,
                 ibuf_a, ibuf_b, wsem, isem, osem):
    wid = lax.axis_index("s") * NC + lax.axis_index("c")
    d0 = wid * DPW
    full = d0 + DPW <= D

    pltpu.async_copy(wT_hbm.at[pl.ds(d0, DPW)],
                     wbuf.reshape(DPW, NB), wsem).wait()
    wrows = [wbuf.reshape(DPW, NB).at[dd] for dd in range(DPW)]

    def idx_issue(s, ibuf):
        pltpu.async_copy(idxT_hbm.at[s], ibuf, isem)

    def idx_drain(s, ibuf):
        pltpu.make_async_copy(idxT_hbm.at[s], ibuf, isem).wait()

    def out_pairs(s, obuf):
        view = obuf.reshape(DPW, NB)
        return ((view, r_hbm.at[s, pl.ds(d0, DPW)]),
                (view.at[pl.ds(0, 8)], r_hbm.at[s, pl.ds(d0, 8)]))

    def out_issue(s, obuf):
        fp, pp = out_pairs(s, obuf)

        def _f():
            pltpu.async_copy(*fp, osem)

        def _p():
            pltpu.async_copy(*pp, osem)

        pl.when(full)(_f)
        pl.when(jnp.logical_not(full))(_p)

    def out_drain(s, obuf):
        fp, pp = out_pairs(s, obuf)

        def _f():
            pltpu.make_async_copy(*fp, osem).wait()

        def _p():
            pltpu.make_async_copy(*pp, osem).wait()

        pl.when(full)(_f)
        pl.when(jnp.logical_not(full))(_p)

    def compute(ibuf, obuf):
        def jbody(j, carry):
            for u in range(8):
                v16 = ibuf[pl.ds(j * 128 + u * 16, 16)]
                for dd in range(DPW):
                    g = plsc.load_gather(wrows[dd], [v16])
                    obuf[8 * dd + j, pl.ds(u * 16, 16)] = g
            return carry
        lax.fori_loop(0, 8, jbody, 0)

    def step(s, ibuf_cur, ibuf_nxt, obuf_cur, first, last_pair):
        if not first:
            idx_drain(s, ibuf_cur)
            pl.when(s + 1 <= S - 1)(lambda: idx_issue(s + 1, ibuf_nxt))
            pl.when(s >= 2)(lambda: out_drain(s - 2, obuf_cur))
        compute(ibuf_cur, obuf_cur)
        out_issue(s, obuf_cur)

    pltpu.sync_copy(idxT_hbm.at[0], ibuf_a)
    idx_issue(1, ibuf_b)
    compute(ibuf_a, obuf_a)
    out_issue(0, obuf_a)
    idx_drain(1, ibuf_b)
    idx_issue(2, ibuf_a)
    compute(ibuf_b, obuf_b)
    out_issue(1, obuf_b)

    def mid(i, carry):
        s0 = 2 * i
        step(s0, ibuf_a, ibuf_b, obuf_a, False, False)
        step(s0 + 1, ibuf_b, ibuf_a, obuf_b, False, False)
        return carry

    lax.fori_loop(1, S // 2, mid, 0)

    out_drain(S - 2, obuf_a)
    out_drain(S - 1, obuf_b)


def kernel(idx, w):
    wT = jnp.pad(w.T, ((0, 1024 - D), (0, NB - D)))
    idxT = idx.T.astype(jnp.int32)
    r = _lane_kernel(idxT, wT)
    return jnp.transpose(r, (2, 0, 1))


# speedup vs baseline: 3.2772x; 2.8594x over previous
import functools

import jax
import jax.numpy as jnp
from jax import lax
from jax.experimental import pallas as pl
from jax.experimental.pallas import tpu as pltpu
from jax.experimental.pallas import tpu_sc as plsc

NB = 1024
S = 50
D = 1000
DPW = 32
NC, NS_ = 2, 16
NW = NC * NS_

_mesh = plsc.VectorSubcoreMesh(core_axis_name="c", subcore_axis_name="s")


@functools.partial(
    pl.kernel,
    mesh=_mesh,
    out_type=jax.ShapeDtypeStruct((S, D, NB), jnp.float32),
    scratch_types=[
        pltpu.VMEM((DPW * 8, 128), jnp.float32),
        pltpu.VMEM((DPW * 8, 128), jnp.float32),
        pltpu.VMEM((DPW * 8, 128), jnp.float32),
        pltpu.VMEM((NB,), jnp.int32),
        pltpu.VMEM((NB,), jnp.int32),
        pltpu.SemaphoreType.DMA,
        pltpu.SemaphoreType.DMA,
        pltpu.SemaphoreType.DMA,
    ],
    compiler_params=pltpu.CompilerParams(needs_layout_passes=False),
)
def _lane_kernel(idxT_hbm, wT_hbm, r_hbm, wbuf, obuf_a, obuf_b,
                 ibuf_a, ibuf_b, wsem, isem, osem):
    wid = lax.axis_index("s") * NC + lax.axis_index("c")
    d0 = wid * DPW
    full = d0 + DPW <= D

    pltpu.async_copy(wT_hbm.at[pl.ds(d0, DPW)],
                     wbuf.reshape(DPW, NB), wsem).wait()
    wrows = [wbuf.reshape(DPW, NB).at[dd] for dd in range(DPW)]

    def idx_issue(s, ibuf):
        pltpu.async_copy(idxT_hbm.at[s], ibuf, isem)

    def idx_drain(s, ibuf):
        pltpu.make_async_copy(idxT_hbm.at[s], ibuf, isem).wait()

    def out_pairs(s, obuf):
        view = obuf.reshape(DPW, NB)
        return ((view, r_hbm.at[s, pl.ds(d0, DPW)]),
                (view.at[pl.ds(0, 8)], r_hbm.at[s, pl.ds(d0, 8)]))

    def out_issue(s, obuf):
        fp, pp = out_pairs(s, obuf)

        def _f():
            pltpu.async_copy(*fp, osem)

        def _p():
            pltpu.async_copy(*pp, osem)

        pl.when(full)(_f)
        pl.when(jnp.logical_not(full))(_p)

    def out_drain(s, obuf):
        fp, pp = out_pairs(s, obuf)

        def _f():
            pltpu.make_async_copy(*fp, osem).wait()

        def _p():
            pltpu.make_async_copy(*pp, osem).wait()

        pl.when(full)(_f)
        pl.when(jnp.logical_not(full))(_p)

    def compute(ibuf, obuf):
        def jbody(j, carry):
            vs = [ibuf[pl.ds(j * 128 + u * 16, 16)] for u in range(8)]
            for dd in range(DPW):
                gs = [plsc.load_gather(wrows[dd], [vs[u]]) for u in range(8)]
                for u in range(8):
                    obuf[8 * dd + j, pl.ds(u * 16, 16)] = gs[u]
            return carry
        lax.fori_loop(0, 8, jbody, 0)

    def step(s, ibuf_cur, ibuf_nxt, obuf_cur, first, last_pair):
        if not first:
            idx_drain(s, ibuf_cur)
            pl.when(s + 1 <= S - 1)(lambda: idx_issue(s + 1, ibuf_nxt))
            pl.when(s >= 2)(lambda: out_drain(s - 2, obuf_cur))
        compute(ibuf_cur, obuf_cur)
        out_issue(s, obuf_cur)

    pltpu.sync_copy(idxT_hbm.at[0], ibuf_a)
    idx_issue(1, ibuf_b)
    compute(ibuf_a, obuf_a)
    out_issue(0, obuf_a)
    idx_drain(1, ibuf_b)
    idx_issue(2, ibuf_a)
    compute(ibuf_b, obuf_b)
    out_issue(1, obuf_b)

    def mid(i, carry):
        s0 = 2 * i
        step(s0, ibuf_a, ibuf_b, obuf_a, False, False)
        step(s0 + 1, ibuf_b, ibuf_a, obuf_b, False, False)
        return carry

    lax.fori_loop(1, S // 2, mid, 0)

    out_drain(S - 2, obuf_a)
    out_drain(S - 1, obuf_b)


def kernel(idx, w):
    wT = jnp.pad(w.T, ((0, 1024 - D), (0, NB - D)))
    idxT = idx.T.astype(jnp.int32)
    r = _lane_kernel(idxT, wT)
    return jnp.transpose(r, (2, 0, 1))
